# baseline (speedup 1.0000x reference)
import functools

import jax
import jax.numpy as jnp
from jax import lax
from jax.experimental import pallas as pl
from jax.experimental.pallas import tpu as pltpu
from jax.experimental.pallas import tpu_sc as plsc

_LANES = 16


@functools.lru_cache(maxsize=None)
def _make_lookup(V, D, B, H):
    info = plsc.get_sparse_core_info()
    nw = info.num_cores * info.num_subcores
    DG = D // 8
    BT = B // 128
    assert D % 8 == 0 and B % (128 * nw) == 0 and H % 2 == 0
    bt_w = BT // nw
    b_w = bt_w * 128

    mesh = plsc.VectorSubcoreMesh(core_axis_name="c", subcore_axis_name="s")

    @functools.partial(
        pl.kernel,
        mesh=mesh,
        out_type=jax.ShapeDtypeStruct((H, DG, BT * 8, 128), jnp.float32),
        scratch_types=[
            pltpu.VMEM((b_w, H), jnp.int32),
            pltpu.VMEM((H, b_w), jnp.int32),
            [pltpu.VMEM((bt_w * 128, D), jnp.float32) for _ in range(2)],
            [pltpu.VMEM((DG, bt_w * 8, 128), jnp.float32) for _ in range(2)],
            pltpu.SemaphoreType.DMA,
            pltpu.SemaphoreType.DMA,
            pltpu.SemaphoreType.DMA,
            pltpu.SemaphoreType.DMA,
        ],
        compiler_params=pltpu.CompilerParams(
            use_tc_tiling_on_sc=False, needs_layout_passes=False
        ),
    )
    def lookup_kernel(table_hbm, idx_hbm, out_hbm, idxs_v, idx_v, gbufs,
                      tbufs, gsem_a, gsem_b, wsem_a, wsem_b):
        wid = lax.axis_index("s") * info.num_cores + lax.axis_index("c")
        b0 = wid * b_w
        gsems = (gsem_a, gsem_b)
        wsems = (wsem_a, wsem_b)

        pltpu.sync_copy(idx_hbm.at[pl.ds(b0, b_w)], idxs_v)

        iota = lax.iota(jnp.int32, _LANES)
        bls = [iota + (blk * _LANES) for blk in range(128 // _LANES)]

        @plsc.parallel_loop(0, H, unroll=2)
        def _(h):
            h_splat = jnp.full((_LANES,), h, jnp.int32)
            for blk in range(b_w // _LANES):
                bl = iota + (blk * _LANES)
                vals = plsc.load_gather(idxs_v, [bl, h_splat])
                idx_v[h, pl.ds(blk * _LANES, _LANES)] = vals

        def fire_gathers(h, s):
            return [
                pltpu.async_copy(
                    table_hbm.at[idx_v.at[h, pl.ds(j * 128, 128)]],
                    gbufs[s].at[pl.ds(j * 128, 128)],
                    gsems[s],
                )
                for j in range(bt_w)
            ]

        def wait_gathers(s):
            dummy = idx_v.at[0, pl.ds(0, 128)]
            for j in range(bt_w):
                pltpu.make_async_copy(
                    table_hbm.at[dummy], gbufs[s].at[pl.ds(j * 128, 128)],
                    gsems[s],
                ).wait()

        def out_slice(h):
            return out_hbm.at[h, :, pl.ds(wid * bt_w * 8, bt_w * 8)]

        i16 = lax.iota(jnp.int32, _LANES)
        dg_lo = i16 // 8
        ds16 = i16 % 8

        def process(h, s):
            wait_gathers(s)
            tb = tbufs[s]
            gb = gbufs[s]

            @plsc.parallel_loop(0, b_w, unroll=4)
            def _(r):
                t_vec = ds16 + ((r // 128) * 8)
                bl_splat = jnp.full((_LANES,), r % 128, jnp.int32)
                for half in range(2):
                    vals = gb[r, pl.ds(half * _LANES, _LANES)]
                    plsc.store_scatter(
                        tb, [dg_lo + 2 * half, t_vec, bl_splat], vals
                    )

            pltpu.async_copy(tb, out_slice(h), wsems[s])

        def drain_write(s):
            pltpu.make_async_copy(tbufs[s], out_slice(0), wsems[s]).wait()

        fire_gathers(0, 0)

        def body(hh, carry):
            h0 = 2 * hh
            h1 = h0 + 1
            fire_gathers(h1, 1)

            @pl.when(hh >= 1)
            def _():
                drain_write(0)

            process(h0, 0)

            @pl.when(hh < H // 2 - 1)
            def _():
                fire_gathers(h0 + 2, 0)

            @pl.when(hh >= 1)
            def _():
                drain_write(1)

            process(h1, 1)
            return carry

        lax.fori_loop(0, H // 2, body, 0)
        drain_write(0)
        drain_write(1)

    return lookup_kernel


def kernel(tactic_labels, table):
    B, H = tactic_labels.shape
    V, D = table.shape
    idx = tactic_labels.astype(jnp.int32)
    o5 = _make_lookup(V, D, B, H)(table.astype(jnp.float32), idx)
    o5 = o5.reshape(o5.shape[0], o5.shape[1], B // 128, 8, 128)
    return o5.transpose(2, 4, 0, 1, 3).reshape(B, H, D)


# speedup vs baseline: 1.0334x; 1.0334x over previous
import functools

import jax
import jax.numpy as jnp
from jax import lax
from jax.experimental import pallas as pl
from jax.experimental.pallas import tpu as pltpu
from jax.experimental.pallas import tpu_sc as plsc

_LANES = 16


@functools.lru_cache(maxsize=None)
def _make_lookup(V, D, B, H):
    info = plsc.get_sparse_core_info()
    nw = info.num_cores * info.num_subcores
    DG = D // 8
    BT = B // 128
    assert D % 8 == 0 and B % (128 * nw) == 0 and H % 2 == 0
    bt_w = BT // nw
    b_w = bt_w * 128

    mesh = plsc.VectorSubcoreMesh(core_axis_name="c", subcore_axis_name="s")

    @functools.partial(
        pl.kernel,
        mesh=mesh,
        out_type=jax.ShapeDtypeStruct((H, DG, BT * 8, 128), jnp.float32),
        scratch_types=[
            pltpu.VMEM((b_w, H), jnp.int32),
            pltpu.VMEM((H, b_w), jnp.int32),
            [pltpu.VMEM((bt_w * 128, D), jnp.float32) for _ in range(2)],
            [pltpu.VMEM((DG, bt_w * 8, 128), jnp.float32) for _ in range(2)],
            pltpu.SemaphoreType.DMA,
            pltpu.SemaphoreType.DMA,
            pltpu.SemaphoreType.DMA,
            pltpu.SemaphoreType.DMA,
        ],
        compiler_params=pltpu.CompilerParams(
            use_tc_tiling_on_sc=False, needs_layout_passes=False
        ),
    )
    def lookup_kernel(table_hbm, idx_hbm, out_hbm, idxs_v, idx_v, gbufs,
                      tbufs, gsem_a, gsem_b, wsem_a, wsem_b):
        wid = lax.axis_index("s") * info.num_cores + lax.axis_index("c")
        b0 = wid * b_w
        gsems = (gsem_a, gsem_b)
        wsems = (wsem_a, wsem_b)

        pltpu.sync_copy(idx_hbm.at[pl.ds(b0, b_w)], idxs_v)

        iota = lax.iota(jnp.int32, _LANES)
        bls = [iota + (blk * _LANES) for blk in range(128 // _LANES)]

        @plsc.parallel_loop(0, H, unroll=2)
        def _(h):
            h_splat = jnp.full((_LANES,), h, jnp.int32)
            for blk in range(b_w // _LANES):
                bl = iota + (blk * _LANES)
                vals = plsc.load_gather(idxs_v, [bl, h_splat])
                idx_v[h, pl.ds(blk * _LANES, _LANES)] = vals

        def fire_gathers(h, s):
            return [
                pltpu.async_copy(
                    table_hbm.at[idx_v.at[h, pl.ds(j * 128, 128)]],
                    gbufs[s].at[pl.ds(j * 128, 128)],
                    gsems[s],
                )
                for j in range(bt_w)
            ]

        def wait_gathers(s):
            dummy = idx_v.at[0, pl.ds(0, 128)]
            for j in range(bt_w):
                pltpu.make_async_copy(
                    table_hbm.at[dummy],
                    gbufs[s].at[pl.ds(j * 128, 128)],
                    gsems[s],
                ).wait()

        def out_slice(h):
            return out_hbm.at[h, :, pl.ds(wid * bt_w * 8, bt_w * 8)]

        def process(h, s):
            wait_gathers(s)
            tb = tbufs[s]
            gb = gbufs[s]

            @plsc.parallel_loop(0, bt_w * 8, unroll=4)
            def _(t):
                j8 = (t // 8) * 128
                ds_ = t % 8
                for dg in range(DG):
                    d_splat = jnp.full((_LANES,), dg * 8 + ds_, jnp.int32)
                    for blk in range(128 // _LANES):
                        vals = plsc.load_gather(gb, [j8 + bls[blk], d_splat])
                        tb[dg, t, pl.ds(blk * _LANES, _LANES)] = vals

            pltpu.async_copy(tb, out_slice(h), wsems[s])

        def drain_write(s):
            pltpu.make_async_copy(tbufs[s], out_slice(0), wsems[s]).wait()

        fire_gathers(0, 0)

        def body(hh, carry):
            h0 = 2 * hh
            h1 = h0 + 1
            fire_gathers(h1, 1)

            @pl.when(hh >= 1)
            def _():
                drain_write(0)

            process(h0, 0)

            @pl.when(hh < H // 2 - 1)
            def _():
                fire_gathers(h0 + 2, 0)

            @pl.when(hh >= 1)
            def _():
                drain_write(1)

            process(h1, 1)
            return carry

        lax.fori_loop(0, H // 2, body, 0)
        drain_write(0)
        drain_write(1)

    return lookup_kernel


def kernel(tactic_labels, table):
    B, H = tactic_labels.shape
    V, D = table.shape
    idx = tactic_labels.astype(jnp.int32)
    o5 = _make_lookup(V, D, B, H)(table.astype(jnp.float32), idx)
    o5 = o5.reshape(o5.shape[0], o5.shape[1], B // 128, 8, 128)
    return o5.transpose(2, 4, 0, 1, 3).reshape(B, H, D)
